# baseline (speedup 1.0000x reference)
import functools

import jax
import jax.numpy as jnp
from jax import lax
from jax.experimental import pallas as pl
from jax.experimental.pallas import tpu as pltpu
from jax.experimental.pallas import tpu_sc as plsc

_B = 64
_T = 128
_D = 768
_V = 50257
_NB = _B * _T
_NC = 2
_NS = 16
_NW = _NC * _NS
_B_PER_W = _NB // _NW
_CHUNK = 64
_NCHUNK = _B_PER_W // _CHUNK
_HEAD = 8192


@functools.partial(
    pl.kernel,
    mesh=plsc.VectorSubcoreMesh(core_axis_name="c", subcore_axis_name="s"),
    out_type=jax.ShapeDtypeStruct((_NB, _D), jnp.float32),
    scratch_types=[
        pltpu.VMEM((_NCHUNK, _CHUNK), jnp.int32),
        pltpu.VMEM((_CHUNK, _D), jnp.float32),
        pltpu.VMEM((_CHUNK, _D), jnp.float32),
        pltpu.SemaphoreType.DMA,
        pltpu.SemaphoreType.DMA,
    ],
)
def _gather_rows(table_hbm, idx_hbm, out_hbm, idx_v, buf0, buf1, sem0, sem1):
    wid = lax.axis_index("s") * _NC + lax.axis_index("c")
    base = wid * _B_PER_W
    pltpu.sync_copy(idx_hbm.at[wid], idx_v)

    bufs = (buf0, buf1)
    sems = (sem0, sem1)

    def start(c):
        return pltpu.async_copy(
            table_hbm.at[idx_v.at[c]], bufs[c % 2], sems[c % 2]
        )

    cur = start(0)
    for c in range(_NCHUNK):
        nxt = start(c + 1) if c + 1 < _NCHUNK else None
        cur.wait()
        pltpu.sync_copy(
            bufs[c % 2], out_hbm.at[pl.ds(base + c * _CHUNK, _CHUNK)]
        )
        cur = nxt


def kernel(time_series_data, input_ids, word_embedding, pred_len=96, seq_len=512):
    ids = input_ids.reshape(_NW, _NCHUNK, _CHUNK)
    zero = time_series_data[0, 0, 0] * 0.0
    head = word_embedding[:_HEAD] + zero
    ids_dep, head_dep = lax.optimization_barrier((ids, head))
    flat = _gather_rows(word_embedding, ids_dep)
    tail = word_embedding[_HEAD:] + zero
    table_out = jnp.concatenate([head_dep, tail], axis=0)
    return (table_out, flat.reshape(_B, _T, _D))


# speedup vs baseline: 1.7706x; 1.7706x over previous
import functools

import jax
import jax.numpy as jnp
from jax import lax
from jax.experimental import pallas as pl
from jax.experimental.pallas import tpu as pltpu
from jax.experimental.pallas import tpu_sc as plsc

_B = 64
_T = 128
_D = 768
_NB = _B * _T
_NC = 2
_NS = 16
_NW = _NC * _NS
_B_PER_W = _NB // _NW
_CHUNK = 64
_NCHUNK = _B_PER_W // _CHUNK
_ROWS_PER_W = _B_PER_W // _T


@functools.partial(
    pl.kernel,
    mesh=plsc.VectorSubcoreMesh(core_axis_name="c", subcore_axis_name="s"),
    out_type=jax.ShapeDtypeStruct((_NB, _D), jnp.float32),
    scratch_types=[
        pltpu.VMEM((_ROWS_PER_W, _T), jnp.int32),
        pltpu.VMEM((_CHUNK, _D), jnp.float32),
        pltpu.VMEM((_CHUNK, _D), jnp.float32),
        pltpu.SemaphoreType.DMA,
        pltpu.SemaphoreType.DMA,
    ],
)
def _gather_rows(table_hbm, idx_hbm, out_hbm, idx_v, buf0, buf1, sem0, sem1):
    wid = lax.axis_index("s") * _NC + lax.axis_index("c")
    base = wid * _B_PER_W
    for r in range(_ROWS_PER_W):
        pltpu.sync_copy(idx_hbm.at[wid * _ROWS_PER_W + r], idx_v.at[r])

    bufs = (buf0, buf1)
    sems = (sem0, sem1)

    def start(c):
        idx_slice = idx_v.at[c // 2, pl.ds((c % 2) * _CHUNK, _CHUNK)]
        return pltpu.async_copy(
            table_hbm.at[idx_slice], bufs[c % 2], sems[c % 2]
        )

    cur = start(0)
    for c in range(_NCHUNK):
        nxt = start(c + 1) if c + 1 < _NCHUNK else None
        cur.wait()
        pltpu.sync_copy(
            bufs[c % 2], out_hbm.at[pl.ds(base + c * _CHUNK, _CHUNK)]
        )
        cur = nxt


def kernel(time_series_data, input_ids, word_embedding, pred_len=96, seq_len=512):
    flat = _gather_rows(word_embedding, input_ids)
    zero = time_series_data[0, 0, 0] * 0.0
    return (word_embedding + zero, flat.reshape(_B, _T, _D))


# speedup vs baseline: 1.7974x; 1.0152x over previous
import functools

import jax
import jax.numpy as jnp
from jax import lax
from jax.experimental import pallas as pl
from jax.experimental.pallas import tpu as pltpu
from jax.experimental.pallas import tpu_sc as plsc

_B = 64
_T = 128
_D = 768
_NB = _B * _T
_NC = 2
_NS = 16
_NW = _NC * _NS
_B_PER_W = _NB // _NW
_CHUNK = 64
_NCHUNK = _B_PER_W // _CHUNK
_ROWS_PER_W = _B_PER_W // _T


@functools.partial(
    pl.kernel,
    mesh=plsc.VectorSubcoreMesh(core_axis_name="c", subcore_axis_name="s"),
    out_type=jax.ShapeDtypeStruct((_NB, _D), jnp.float32),
    scratch_types=[
        pltpu.VMEM((_ROWS_PER_W, _T), jnp.int32),
        pltpu.VMEM((_CHUNK, _D), jnp.float32),
        pltpu.VMEM((_CHUNK, _D), jnp.float32),
        pltpu.SemaphoreType.DMA,
        pltpu.SemaphoreType.DMA,
    ],
)
def _gather_rows(table_hbm, idx_hbm, out_hbm, idx_v, buf0, buf1, sem0, sem1):
    wid = lax.axis_index("s") * _NC + lax.axis_index("c")
    base = wid * _B_PER_W
    for r in range(_ROWS_PER_W):
        pltpu.sync_copy(idx_hbm.at[wid * _ROWS_PER_W + r], idx_v.at[r])

    bufs = (buf0, buf1)
    sems = (sem0, sem1)

    def start(c):
        idx_slice = idx_v.at[c // 2, pl.ds((c % 2) * _CHUNK, _CHUNK)]
        return pltpu.async_copy(
            table_hbm.at[idx_slice], bufs[c % 2], sems[c % 2]
        )

    cur = start(0)
    for c in range(_NCHUNK):
        nxt = start(c + 1) if c + 1 < _NCHUNK else None
        cur.wait()
        pltpu.sync_copy(
            bufs[c % 2], out_hbm.at[pl.ds(base + c * _CHUNK, _CHUNK)]
        )
        cur = nxt


def kernel(time_series_data, input_ids, word_embedding, pred_len=96, seq_len=512):
    flat = _gather_rows(word_embedding, input_ids)
    table_out = word_embedding + word_embedding * 0.0
    return (table_out, flat.reshape(_B, _T, _D))


# speedup vs baseline: 1.7994x; 1.0011x over previous
import functools

import jax
import jax.numpy as jnp
from jax import lax
from jax.experimental import pallas as pl
from jax.experimental.pallas import tpu as pltpu
from jax.experimental.pallas import tpu_sc as plsc

_B = 64
_T = 128
_D = 768
_NB = _B * _T
_NC = 2
_NS = 16
_NW = _NC * _NS
_B_PER_W = _NB // _NW
_CHUNK = 64
_NCHUNK = _B_PER_W // _CHUNK
_ROWS_PER_W = _B_PER_W // _T


@functools.partial(
    pl.kernel,
    mesh=plsc.VectorSubcoreMesh(core_axis_name="c", subcore_axis_name="s"),
    out_type=jax.ShapeDtypeStruct((_NB, _D), jnp.float32),
    scratch_types=[
        pltpu.VMEM((_ROWS_PER_W, _T), jnp.int32),
        pltpu.VMEM((_CHUNK, _D), jnp.float32),
        pltpu.VMEM((_CHUNK, _D), jnp.float32),
        pltpu.SemaphoreType.DMA,
        pltpu.SemaphoreType.DMA,
    ],
)
def _gather_rows(table_hbm, idx_hbm, out_hbm, idx_v, buf0, buf1, sem0, sem1):
    wid = lax.axis_index("s") * _NC + lax.axis_index("c")
    base = wid * _B_PER_W
    for r in range(_ROWS_PER_W):
        pltpu.sync_copy(idx_hbm.at[wid * _ROWS_PER_W + r], idx_v.at[r])

    bufs = (buf0, buf1)
    sems = (sem0, sem1)

    def start(row, b):
        idx_slice = idx_v.at[row, pl.ds(b * _CHUNK, _CHUNK)]
        return pltpu.async_copy(table_hbm.at[idx_slice], bufs[b], sems[b])

    def wait(b):
        pltpu.make_async_copy(
            table_hbm.at[pl.ds(0, _CHUNK)], bufs[b], sems[b]
        ).wait()

    start(0, 0)
    start(0, 1)

    def body(g, carry):
        for b in range(2):
            wait(b)
            pltpu.sync_copy(
                bufs[b],
                out_hbm.at[pl.ds(base + (2 * g + b) * _CHUNK, _CHUNK)],
            )

            @pl.when(g + 1 < _ROWS_PER_W)
            def _():
                start(g + 1, b)

        return carry

    lax.fori_loop(0, _ROWS_PER_W, body, 0)


def kernel(time_series_data, input_ids, word_embedding, pred_len=96, seq_len=512):
    flat = _gather_rows(word_embedding, input_ids)
    table_out = word_embedding + word_embedding * 0.0
    return (table_out, flat.reshape(_B, _T, _D))
